# baseline (speedup 1.0000x reference)
import functools

import jax
import jax.numpy as jnp
from jax.experimental import pallas as pl
from jax.experimental.pallas import tpu as pltpu


def _rup(x, m):
    return ((x + m - 1) // m) * m


def _probe3d_kernel(b_ref, x_ref, w_ref, o_ref, *, F1, F2, F1p, F2p):
    x = x_ref[...].astype(jnp.float32)
    if F1p != F1 or F2p != F2:
        ok = jnp.ones((), jnp.bool_)
        if F1p != F1:
            r = jax.lax.broadcasted_iota(jnp.int32, x.shape, 1)
            ok = ok & (r < F1)
        if F2p != F2:
            c = jax.lax.broadcasted_iota(jnp.int32, x.shape, 2)
            ok = ok & (c < F2)
        x = jnp.where(ok, x, 0.0)
    part = jnp.sum(x * w_ref[...], axis=1)
    o_ref[...] = (
        jnp.sum(part, axis=-1, keepdims=True) + b_ref[0, 0]
    ).astype(o_ref.dtype)


def _probe2d_kernel(b_ref, x_ref, w_ref, o_ref, *, D, Dp):
    x = x_ref[...].astype(jnp.float32)
    if Dp != D:
        col = jax.lax.broadcasted_iota(jnp.int32, x.shape, 1)
        x = jnp.where(col < D, x, 0.0)
    row = jnp.sum(x * w_ref[...], axis=-1, keepdims=True)
    o_ref[...] = (row + b_ref[0, 0]).astype(o_ref.dtype)


def _pick_tb(B, bytes_per_row, sub):
    max_tb = max(sub, (40 * 1024 * 1024 // (2 * bytes_per_row)) // sub * sub)
    for tb in range(min(max_tb, B, 128), sub - 1, -sub):
        if B % tb == 0 and (B // tb) % 2 == 0:
            return tb
    TB = min(max_tb, _rup(B, sub))
    if -(-B // TB) < 2 and B >= 2 * sub:
        TB = _rup(-(-B // 2), sub)
    return TB


def kernel(x, weight, bias):
    B = x.shape[0]
    bias_smem = jnp.asarray(bias, jnp.float32).reshape(1, 1)
    itemsize = jnp.dtype(x.dtype).itemsize
    sub = max(8, 32 // max(1, itemsize))

    if x.ndim >= 3:
        F1 = 1
        for s in x.shape[1:-1]:
            F1 *= s
        F2 = x.shape[-1]
        x3 = x.reshape(B, F1, F2)
        F1p, F2p = _rup(F1, 8), _rup(F2, 128)

        w2 = weight.reshape(F1, F2).astype(jnp.float32)
        if (F1p, F2p) != (F1, F2):
            w2 = jnp.pad(w2, ((0, F1p - F1), (0, F2p - F2)))

        TB = _pick_tb(B, F1p * F2p * max(4, itemsize), sub)
        out = pl.pallas_call(
            functools.partial(_probe3d_kernel, F1=F1, F2=F2, F1p=F1p, F2p=F2p),
            out_shape=jax.ShapeDtypeStruct((B, 1), x.dtype),
            grid=(-(-B // TB),),
            in_specs=[
                pl.BlockSpec(memory_space=pltpu.SMEM),
                pl.BlockSpec((TB, F1p, F2p), lambda i: (i, 0, 0)),
                pl.BlockSpec((F1p, F2p), lambda i: (0, 0)),
            ],
            out_specs=pl.BlockSpec((TB, 1), lambda i: (i, 0)),
            compiler_params=pltpu.CompilerParams(
                dimension_semantics=("parallel",),
                vmem_limit_bytes=56 * 1024 * 1024,
            ),
        )(bias_smem, x3, w2)
    else:
        x2d = x.reshape(B, -1)
        D = x2d.shape[1]
        Dp = _rup(D, 128)
        w_row = weight.reshape(1, D).astype(jnp.float32)
        if Dp != D:
            w_row = jnp.pad(w_row, ((0, 0), (0, Dp - D)))
        TB = _pick_tb(B, Dp * max(4, itemsize), sub)
        out = pl.pallas_call(
            functools.partial(_probe2d_kernel, D=D, Dp=Dp),
            out_shape=jax.ShapeDtypeStruct((B, 1), x2d.dtype),
            grid=(-(-B // TB),),
            in_specs=[
                pl.BlockSpec(memory_space=pltpu.SMEM),
                pl.BlockSpec((TB, Dp), lambda i: (i, 0)),
                pl.BlockSpec((1, Dp), lambda i: (0, 0)),
            ],
            out_specs=pl.BlockSpec((TB, 1), lambda i: (i, 0)),
            compiler_params=pltpu.CompilerParams(
                dimension_semantics=("parallel",),
                vmem_limit_bytes=56 * 1024 * 1024,
            ),
        )(bias_smem, x2d, w_row)

    return jnp.squeeze(out)


# speedup vs baseline: 1.1406x; 1.1406x over previous
import functools

import jax
import jax.numpy as jnp
from jax.experimental import pallas as pl
from jax.experimental.pallas import tpu as pltpu


def _rup(x, m):
    return ((x + m - 1) // m) * m


def _probe3d_kernel(b_ref, x_ref, w_ref, o_ref, *, F1, F2, F1p, F2p):
    x = x_ref[...].astype(jnp.float32)
    if F1p != F1 or F2p != F2:
        ok = jnp.ones((), jnp.bool_)
        if F1p != F1:
            r = jax.lax.broadcasted_iota(jnp.int32, x.shape, 1)
            ok = ok & (r < F1)
        if F2p != F2:
            c = jax.lax.broadcasted_iota(jnp.int32, x.shape, 2)
            ok = ok & (c < F2)
        x = jnp.where(ok, x, 0.0)
    part = jnp.sum(x * w_ref[...], axis=1)
    o_ref[...] = (
        jnp.sum(part, axis=-1, keepdims=True) + b_ref[0, 0]
    ).astype(o_ref.dtype)


def _probe2d_kernel(b_ref, x_ref, w_ref, o_ref, *, D, Dp):
    x = x_ref[...].astype(jnp.float32)
    if Dp != D:
        col = jax.lax.broadcasted_iota(jnp.int32, x.shape, 1)
        x = jnp.where(col < D, x, 0.0)
    row = jnp.sum(x * w_ref[...], axis=-1, keepdims=True)
    o_ref[...] = (row + b_ref[0, 0]).astype(o_ref.dtype)


def _pick_tb(B, bytes_per_row, sub):
    max_tb = max(sub, (40 * 1024 * 1024 // (2 * bytes_per_row)) // sub * sub)
    for tb in range(min(max_tb, B, 256), sub - 1, -sub):
        if B % tb == 0 and (B // tb) % 2 == 0:
            return tb
    TB = min(max_tb, _rup(B, sub))
    if -(-B // TB) < 2 and B >= 2 * sub:
        TB = _rup(-(-B // 2), sub)
    return TB


def kernel(x, weight, bias):
    B = x.shape[0]
    bias_smem = jnp.asarray(bias, jnp.float32).reshape(1, 1)
    itemsize = jnp.dtype(x.dtype).itemsize
    sub = max(8, 32 // max(1, itemsize))

    if x.ndim >= 3:
        F1 = 1
        for s in x.shape[1:-1]:
            F1 *= s
        F2 = x.shape[-1]
        x3 = x.reshape(B, F1, F2)
        F1p, F2p = _rup(F1, 8), _rup(F2, 128)

        w2 = weight.reshape(F1, F2).astype(jnp.float32)
        if (F1p, F2p) != (F1, F2):
            w2 = jnp.pad(w2, ((0, F1p - F1), (0, F2p - F2)))

        TB = _pick_tb(B, F1p * F2p * max(4, itemsize), sub)
        out = pl.pallas_call(
            functools.partial(_probe3d_kernel, F1=F1, F2=F2, F1p=F1p, F2p=F2p),
            out_shape=jax.ShapeDtypeStruct((B, 1), x.dtype),
            grid=(-(-B // TB),),
            in_specs=[
                pl.BlockSpec(memory_space=pltpu.SMEM),
                pl.BlockSpec((TB, F1p, F2p), lambda i: (i, 0, 0)),
                pl.BlockSpec((F1p, F2p), lambda i: (0, 0)),
            ],
            out_specs=pl.BlockSpec((TB, 1), lambda i: (i, 0)),
            compiler_params=pltpu.CompilerParams(
                dimension_semantics=("parallel",),
                vmem_limit_bytes=56 * 1024 * 1024,
            ),
        )(bias_smem, x3, w2)
    else:
        x2d = x.reshape(B, -1)
        D = x2d.shape[1]
        Dp = _rup(D, 128)
        w_row = weight.reshape(1, D).astype(jnp.float32)
        if Dp != D:
            w_row = jnp.pad(w_row, ((0, 0), (0, Dp - D)))
        TB = _pick_tb(B, Dp * max(4, itemsize), sub)
        out = pl.pallas_call(
            functools.partial(_probe2d_kernel, D=D, Dp=Dp),
            out_shape=jax.ShapeDtypeStruct((B, 1), x2d.dtype),
            grid=(-(-B // TB),),
            in_specs=[
                pl.BlockSpec(memory_space=pltpu.SMEM),
                pl.BlockSpec((TB, Dp), lambda i: (i, 0)),
                pl.BlockSpec((1, Dp), lambda i: (0, 0)),
            ],
            out_specs=pl.BlockSpec((TB, 1), lambda i: (i, 0)),
            compiler_params=pltpu.CompilerParams(
                dimension_semantics=("parallel",),
                vmem_limit_bytes=56 * 1024 * 1024,
            ),
        )(bias_smem, x2d, w_row)

    return jnp.squeeze(out)


# speedup vs baseline: 1.2117x; 1.0623x over previous
import functools

import jax
import jax.numpy as jnp
from jax.experimental import pallas as pl
from jax.experimental.pallas import tpu as pltpu


def _rup(x, m):
    return ((x + m - 1) // m) * m


def _probe3d_kernel(b_ref, x_ref, w_ref, o_ref, *, F1, F2, F1p, F2p):
    x = x_ref[...].astype(jnp.float32)
    if F1p != F1 or F2p != F2:
        ok = jnp.ones((), jnp.bool_)
        if F1p != F1:
            r = jax.lax.broadcasted_iota(jnp.int32, x.shape, 1)
            ok = ok & (r < F1)
        if F2p != F2:
            c = jax.lax.broadcasted_iota(jnp.int32, x.shape, 2)
            ok = ok & (c < F2)
        x = jnp.where(ok, x, 0.0)
    part = jnp.sum(x * w_ref[...], axis=1)
    o_ref[...] = (
        jnp.sum(part, axis=-1) + b_ref[0, 0]
    ).astype(o_ref.dtype)


def _probe2d_kernel(b_ref, x_ref, w_ref, o_ref, *, D, Dp):
    x = x_ref[...].astype(jnp.float32)
    if Dp != D:
        col = jax.lax.broadcasted_iota(jnp.int32, x.shape, 1)
        x = jnp.where(col < D, x, 0.0)
    row = jnp.sum(x * w_ref[...], axis=-1, keepdims=True)
    o_ref[...] = (row + b_ref[0, 0]).astype(o_ref.dtype)


def _pick_tb(B, bytes_per_row, sub):
    max_tb = max(sub, (40 * 1024 * 1024 // (2 * bytes_per_row)) // sub * sub)
    for tb in range(min(max_tb, B, 256), sub - 1, -sub):
        if B % tb == 0 and (B // tb) % 2 == 0:
            return tb
    TB = min(max_tb, _rup(B, sub))
    if -(-B // TB) < 2 and B >= 2 * sub:
        TB = _rup(-(-B // 2), sub)
    return TB


def kernel(x, weight, bias):
    B = x.shape[0]
    bias_smem = jnp.asarray(bias, jnp.float32).reshape(1, 1)
    itemsize = jnp.dtype(x.dtype).itemsize
    sub = max(8, 32 // max(1, itemsize))

    if x.ndim >= 3:
        F1 = 1
        for s in x.shape[1:-1]:
            F1 *= s
        F2 = x.shape[-1]
        x3 = x.reshape(B, F1, F2)
        F1p, F2p = _rup(F1, 8), _rup(F2, 128)

        w2 = weight.reshape(F1, F2).astype(jnp.float32)
        if (F1p, F2p) != (F1, F2):
            w2 = jnp.pad(w2, ((0, F1p - F1), (0, F2p - F2)))

        TB = _pick_tb(B, F1p * F2p * max(4, itemsize), sub)
        out = pl.pallas_call(
            functools.partial(_probe3d_kernel, F1=F1, F2=F2, F1p=F1p, F2p=F2p),
            out_shape=jax.ShapeDtypeStruct((B,), x.dtype),
            grid=(-(-B // TB),),
            in_specs=[
                pl.BlockSpec(memory_space=pltpu.SMEM),
                pl.BlockSpec((TB, F1p, F2p), lambda i: (i, 0, 0)),
                pl.BlockSpec((F1p, F2p), lambda i: (0, 0)),
            ],
            out_specs=pl.BlockSpec((TB,), lambda i: (i,)),
            compiler_params=pltpu.CompilerParams(
                dimension_semantics=("parallel",),
                vmem_limit_bytes=56 * 1024 * 1024,
            ),
        )(bias_smem, x3, w2)
        return out
    else:
        x2d = x.reshape(B, -1)
        D = x2d.shape[1]
        Dp = _rup(D, 128)
        w_row = weight.reshape(1, D).astype(jnp.float32)
        if Dp != D:
            w_row = jnp.pad(w_row, ((0, 0), (0, Dp - D)))
        TB = _pick_tb(B, Dp * max(4, itemsize), sub)
        out = pl.pallas_call(
            functools.partial(_probe2d_kernel, D=D, Dp=Dp),
            out_shape=jax.ShapeDtypeStruct((B, 1), x2d.dtype),
            grid=(-(-B // TB),),
            in_specs=[
                pl.BlockSpec(memory_space=pltpu.SMEM),
                pl.BlockSpec((TB, Dp), lambda i: (i, 0)),
                pl.BlockSpec((1, Dp), lambda i: (0, 0)),
            ],
            out_specs=pl.BlockSpec((TB, 1), lambda i: (i, 0)),
            compiler_params=pltpu.CompilerParams(
                dimension_semantics=("parallel",),
                vmem_limit_bytes=56 * 1024 * 1024,
            ),
        )(bias_smem, x2d, w_row)

    return jnp.squeeze(out)
